# Initial kernel scaffold; baseline (speedup 1.0000x reference)
import jax
import jax.numpy as jnp
from jax.experimental import pallas as pl


def kernel(W, s, noise, spark_pos, spark_energy, spark_age, branch_rand):
    raise NotImplementedError("write your pallas kernel here")


# speedup vs baseline: 1.1199x; 1.1199x over previous
import jax
import jax.numpy as jnp
from jax.experimental import pallas as pl
from jax.experimental.pallas import tpu as pltpu

_N = 8192
_S = 1024
_BRANCH_PROB = 0.3
_BRANCH_THRESHOLD = 0.6
_DECAY_VIS = 1.0 - 0.001 * 0.02
_DECAY = 1.0 - 0.001
_LR_PATH = 0.02
_NOISE_STD = 0.05
_ENERGY_DECAY = 0.98
_FORCE_STEPS = 5
_ROWS = 256
_NB = _N // _ROWS


def _pass1_body(s0_ref, noise_ref, force_ref, w_ref, wout_ref, s1_ref):
    w = w_ref[...]
    y = jax.lax.dot_general(
        s0_ref[...], w, (((1,), (1,)), ((), ())),
        preferred_element_type=jnp.float32)
    act = jax.nn.sigmoid(y + _NOISE_STD * noise_ref[0])
    s1 = jnp.where(force_ref[0] > 0, 1.0, act)
    s1_ref[...] = s1.reshape(1, 1, _ROWS)
    wout_ref[...] = jnp.clip(w * _DECAY, -2.0, 2.0)


def _pass2_body(pos_ref, perm_ref, w_row_ref, br_ref, le_ref, first_ref,
                last_ref, wdef_ref, out_ref, vis_ref):
    i = pl.program_id(0)
    del wdef_ref
    row = w_row_ref[0]
    positive = jnp.maximum(row, 0.0) + 1e-6
    cols = jax.lax.broadcasted_iota(jnp.int32, (1, _N), 1)
    mx = jnp.max(positive, axis=1, keepdims=True)
    m = jnp.min(jnp.where(positive >= mx, cols, _N), axis=1, keepdims=True)
    onehot = cols == m
    le = le_ref[i]
    bmask = (positive > _BRANCH_THRESHOLD) & (br_ref[0] < _BRANCH_PROB)
    upd = (bmask.astype(jnp.float32) + onehot.astype(jnp.float32)) * le
    edge = (bmask | onehot).astype(jnp.float32)

    @pl.when(first_ref[i] == 1)
    def _():
        out_ref[...] = (row + upd).reshape(1, 1, _N)
        vis_ref[...] = edge

    @pl.when(first_ref[i] == 0)
    def _():
        out_ref[...] += upd.reshape(1, 1, _N)
        vis_ref[...] = jnp.maximum(vis_ref[...], edge)

    @pl.when(last_ref[i] == 1)
    def _():
        decay = jnp.where(vis_ref[...] > 0, _DECAY_VIS, _DECAY)
        out_ref[...] = jnp.clip(out_ref[...] * decay.reshape(1, 1, _N),
                                -2.0, 2.0)


def kernel(W, s, noise, spark_pos, spark_energy, spark_age, branch_rand):
    pos32 = spark_pos.astype(jnp.int32)
    order = jnp.argsort(pos32, stable=True).astype(jnp.int32)
    spos = pos32[order]
    change = (spos[1:] != spos[:-1]).astype(jnp.int32)
    one = jnp.ones((1,), jnp.int32)
    first = jnp.concatenate([one, change])
    last = jnp.concatenate([change, one])
    le = (_LR_PATH * spark_energy)[order]

    young = spark_age < _FORCE_STEPS
    force = jnp.zeros((_N,), jnp.float32).at[pos32].set(
        young.astype(jnp.float32))

    s0 = (s * 0.95).reshape(1, _N)
    noise3 = noise.reshape(_NB, 1, _ROWS)
    force3 = force.astype(jnp.float32).reshape(_NB, 1, _ROWS)

    wdef, s1 = pl.pallas_call(
        _pass1_body,
        grid=(_NB,),
        in_specs=[
            pl.BlockSpec((1, _N), lambda b: (0, 0)),
            pl.BlockSpec((1, 1, _ROWS), lambda b: (b, 0, 0)),
            pl.BlockSpec((1, 1, _ROWS), lambda b: (b, 0, 0)),
            pl.BlockSpec((_ROWS, _N), lambda b: (b, 0)),
        ],
        out_specs=[
            pl.BlockSpec((_ROWS, _N), lambda b: (b, 0)),
            pl.BlockSpec((1, 1, _ROWS), lambda b: (b, 0, 0)),
        ],
        out_shape=[
            jax.ShapeDtypeStruct((_N, _N), jnp.float32),
            jax.ShapeDtypeStruct((_NB, 1, _ROWS), jnp.float32),
        ],
        compiler_params=pltpu.CompilerParams(
            dimension_semantics=("arbitrary",)),
    )(s0, noise3, force3, W)

    w3 = W.reshape(_N, 1, _N)
    br3 = branch_rand.reshape(_S, 1, _N)
    wdef3 = wdef.reshape(_N, 1, _N)

    grid_spec = pltpu.PrefetchScalarGridSpec(
        num_scalar_prefetch=2,
        grid=(_S,),
        in_specs=[
            pl.BlockSpec((1, 1, _N), lambda i, pos, perm: (pos[i], 0, 0)),
            pl.BlockSpec((1, 1, _N), lambda i, pos, perm: (perm[i], 0, 0)),
            pl.BlockSpec(memory_space=pltpu.SMEM),
            pl.BlockSpec(memory_space=pltpu.SMEM),
            pl.BlockSpec(memory_space=pltpu.SMEM),
            pl.BlockSpec(memory_space=pl.MemorySpace.ANY),
        ],
        out_specs=pl.BlockSpec((1, 1, _N), lambda i, pos, perm: (pos[i], 0, 0)),
        scratch_shapes=[pltpu.VMEM((1, _N), jnp.float32)],
    )
    wout3 = pl.pallas_call(
        _pass2_body,
        grid_spec=grid_spec,
        out_shape=jax.ShapeDtypeStruct((_N, 1, _N), jnp.float32),
        input_output_aliases={7: 0},
        compiler_params=pltpu.CompilerParams(
            dimension_semantics=("arbitrary",)),
    )(spos, order, w3, br3, le, first, last, wdef3)

    return (wout3.reshape(_N, _N), s1.reshape(_N),
            spark_energy * _ENERGY_DECAY)


# speedup vs baseline: 2.7679x; 2.4716x over previous
import jax
import jax.numpy as jnp
from jax.experimental import pallas as pl
from jax.experimental.pallas import tpu as pltpu
from jax.experimental.pallas import tpu_sc as plsc

_N = 8192
_S = 1024
_BRANCH_PROB = 0.3
_BRANCH_THRESHOLD = 0.6
_DECAY_VIS = 1.0 - 0.001 * 0.02
_DECAY = 1.0 - 0.001
_LR_PATH = 0.02
_NOISE_STD = 0.05
_ENERGY_DECAY = 0.98
_FORCE_STEPS = 5
_ROWS = 256
_NB = _N // _ROWS

_L = 16
_NW = 32
_GPT = _S // _NW
_CH = _N // _L


def _pass1_body(s0_ref, noise_ref, force_ref, w_ref, wout_ref, s1_ref):
    w = w_ref[...]
    y = jax.lax.dot_general(
        s0_ref[...], w, (((1,), (1,)), ((), ())),
        preferred_element_type=jnp.float32)
    act = jax.nn.sigmoid(y + _NOISE_STD * noise_ref[0])
    s1 = jnp.where(force_ref[0] > 0, 1.0, act)
    s1_ref[...] = s1.reshape(1, 1, _ROWS)
    wout_ref[...] = jnp.clip(w * _DECAY, -2.0, 2.0)


def _sc_body(w_hbm, br_hbm, gpos_hbm, gstart_hbm, gcount_hbm, sperm_hbm,
             sle_hbm, wout_hbm, wrow_v, brrow_v, acca_v, bany_v,
             mpos_v, mstart_v, mcount_v, sperm_v, sle_v):
    def scalar_at(ref, idx):
        return ref[pl.ds(idx, _L)][0]
    lane = jax.lax.iota(jnp.int32, _L)
    wid = jax.lax.axis_index("s") * 2 + jax.lax.axis_index("c")
    base_g = wid * _GPT

    pltpu.sync_copy(gpos_hbm.at[pl.ds(base_g, _GPT)],
                    mpos_v.at[pl.ds(0, _GPT)])
    pltpu.sync_copy(gstart_hbm.at[pl.ds(base_g, _GPT)],
                    mstart_v.at[pl.ds(0, _GPT)])
    pltpu.sync_copy(gcount_hbm.at[pl.ds(base_g, _GPT)],
                    mcount_v.at[pl.ds(0, _GPT)])
    pltpu.sync_copy(sperm_hbm, sperm_v)
    pltpu.sync_copy(sle_hbm, sle_v)

    zeros = jnp.zeros((_L,), jnp.float32)

    def zero_chunk(c, _):
        acca_v[pl.ds(c * _L, _L)] = zeros
        bany_v[pl.ds(c * _L, _L)] = zeros
        return 0

    jax.lax.fori_loop(0, _CH, zero_chunk, 0)

    def group_body(k, _):
        pos_k = scalar_at(mpos_v, k)
        start_k = scalar_at(mstart_v, k)
        count_k = scalar_at(mcount_v, k)

        @pl.when(count_k > 0)
        def _():
            pltpu.sync_copy(w_hbm.at[pos_k], wrow_v)

            def am(c, carry):
                bv, bi = carry
                ch = wrow_v[pl.ds(c * _L, _L)]
                p = jnp.maximum(ch, 0.0) + 1e-6
                gt = p > bv
                bv = jnp.where(gt, p, bv)
                bi = jnp.where(gt, lane + c * _L, bi)
                return bv, bi

            bv, bi = jax.lax.fori_loop(
                0, _CH, am,
                (jnp.full((_L,), -1.0, jnp.float32),
                 jnp.zeros((_L,), jnp.int32)))
            best_v = bv[0]
            best_i = bi[0]
            for l in range(1, _L):
                v = bv[l]
                i = bi[l]
                better = (v > best_v) | ((v == best_v) & (i < best_i))
                best_v = jnp.where(better, v, best_v)
                best_i = jnp.where(better, i, best_i)
            m = best_i

            def member(j, sumle):
                jj = start_k + j
                pj = scalar_at(sperm_v, jj)
                lej = scalar_at(sle_v, jj)
                pltpu.sync_copy(br_hbm.at[pj], brrow_v)

                def mb(c, _2):
                    sl = pl.ds(c * _L, _L)
                    hit = brrow_v[sl] < _BRANCH_PROB
                    acca_v[sl] = acca_v[sl] + jnp.where(hit, lej, 0.0)
                    bany_v[sl] = jnp.maximum(bany_v[sl],
                                             jnp.where(hit, 1.0, 0.0))
                    return 0

                jax.lax.fori_loop(0, _CH, mb, 0)
                return sumle + lej

            sumle = jax.lax.fori_loop(0, count_k, member, jnp.float32(0.0))

            def fin(c, _2):
                sl = pl.ds(c * _L, _L)
                w = wrow_v[sl]
                p = jnp.maximum(w, 0.0) + 1e-6
                strong = p > _BRANCH_THRESHOLD
                oh = (lane + c * _L) == m
                upd = (jnp.where(strong, acca_v[sl], 0.0)
                       + jnp.where(oh, sumle, 0.0))
                vis = (strong & (bany_v[sl] > 0.0)) | oh
                dec = jnp.where(vis, _DECAY_VIS, _DECAY)
                out = jnp.minimum(jnp.maximum((w + upd) * dec, -2.0), 2.0)
                wrow_v[sl] = out
                acca_v[sl] = zeros
                bany_v[sl] = zeros
                return 0

            jax.lax.fori_loop(0, _CH, fin, 0)
            pltpu.sync_copy(wrow_v, wout_hbm.at[pos_k])

        return 0

    jax.lax.fori_loop(0, _GPT, group_body, 0)


def _sc_update(W, branch_rand, gpos, gstart, gcount, sperm, sle, wref):
    mesh = plsc.VectorSubcoreMesh(core_axis_name="c", subcore_axis_name="s")
    run = pl.kernel(
        _sc_body,
        out_type=(),
        mesh=mesh,
        scratch_types=[
            pltpu.VMEM((_N,), jnp.float32),
            pltpu.VMEM((_N,), jnp.float32),
            pltpu.VMEM((_N,), jnp.float32),
            pltpu.VMEM((_N,), jnp.float32),
            pltpu.VMEM((_GPT + _L,), jnp.int32),
            pltpu.VMEM((_GPT + _L,), jnp.int32),
            pltpu.VMEM((_GPT + _L,), jnp.int32),
            pltpu.VMEM((_S + _L,), jnp.int32),
            pltpu.VMEM((_S + _L,), jnp.float32),
        ],
    )
    run(W, branch_rand, gpos, gstart, gcount, sperm, sle, wref)


def kernel(W, s, noise, spark_pos, spark_energy, spark_age, branch_rand):
    pos32 = spark_pos.astype(jnp.int32)
    order = jnp.argsort(pos32, stable=True).astype(jnp.int32)
    spos = pos32[order]
    change = (spos[1:] != spos[:-1]).astype(jnp.int32)
    one = jnp.ones((1,), jnp.int32)
    first = jnp.concatenate([one, change])
    le = (_LR_PATH * spark_energy)[order]

    gid = jnp.cumsum(first) - 1
    arange_s = jnp.arange(_S, dtype=jnp.int32)
    g_count = jnp.zeros((_S,), jnp.int32).at[gid].add(1)
    g_start = jnp.full((_S,), _S, jnp.int32).at[gid].min(arange_s)
    g_start = jnp.where(g_count > 0, g_start, 0)
    g_pos = jnp.zeros((_S,), jnp.int32).at[gid].max(spos)
    zpad_i = jnp.zeros((_L,), jnp.int32)
    gpos_p = jnp.concatenate([g_pos, zpad_i])
    gstart_p = jnp.concatenate([g_start, zpad_i])
    gcount_p = jnp.concatenate([g_count, zpad_i])
    sperm_p = jnp.concatenate([order, zpad_i])
    sle_p = jnp.concatenate([le, jnp.zeros((_L,), jnp.float32)])

    young = spark_age < _FORCE_STEPS
    force = jnp.zeros((_N,), jnp.float32).at[pos32].set(
        young.astype(jnp.float32))

    s0 = (s * 0.95).reshape(1, _N)
    noise3 = noise.reshape(_NB, 1, _ROWS)
    force3 = force.reshape(_NB, 1, _ROWS)

    wdef, s1 = pl.pallas_call(
        _pass1_body,
        grid=(_NB,),
        in_specs=[
            pl.BlockSpec((1, _N), lambda b: (0, 0)),
            pl.BlockSpec((1, 1, _ROWS), lambda b: (b, 0, 0)),
            pl.BlockSpec((1, 1, _ROWS), lambda b: (b, 0, 0)),
            pl.BlockSpec((_ROWS, _N), lambda b: (b, 0)),
        ],
        out_specs=[
            pl.BlockSpec((_ROWS, _N), lambda b: (b, 0)),
            pl.BlockSpec((1, 1, _ROWS), lambda b: (b, 0, 0)),
        ],
        out_shape=[
            jax.ShapeDtypeStruct((_N, _N), jnp.float32),
            jax.ShapeDtypeStruct((_NB, 1, _ROWS), jnp.float32),
        ],
        compiler_params=pltpu.CompilerParams(
            dimension_semantics=("arbitrary",)),
    )(s0, noise3, force3, W)

    wref = jax.new_ref(wdef)
    _sc_update(W, branch_rand, gpos_p, gstart_p, gcount_p, sperm_p, sle_p,
               wref)
    wout = wref[...]

    return (wout, s1.reshape(_N), spark_energy * _ENERGY_DECAY)


# speedup vs baseline: 3.2264x; 1.1656x over previous
import jax
import jax.numpy as jnp
from jax.experimental import pallas as pl
from jax.experimental.pallas import tpu as pltpu
from jax.experimental.pallas import tpu_sc as plsc

_N = 8192
_S = 1024
_BRANCH_PROB = 0.3
_BRANCH_THRESHOLD = 0.6
_DECAY_VIS = 1.0 - 0.001 * 0.02
_DECAY = 1.0 - 0.001
_LR_PATH = 0.02
_NOISE_STD = 0.05
_ENERGY_DECAY = 0.98
_FORCE_STEPS = 5
_ROWS = 256
_NB = _N // _ROWS

_L = 16
_NW = 32
_GPT = _S // _NW
_CH = _N // _L
_U = 8


def _pass1_body(s0_ref, noise_ref, force_ref, w_ref, wout_ref, s1_ref):
    w = w_ref[...]
    y = jax.lax.dot_general(
        s0_ref[...], w, (((1,), (1,)), ((), ())),
        preferred_element_type=jnp.float32)
    act = jax.nn.sigmoid(y + _NOISE_STD * noise_ref[0])
    s1 = jnp.where(force_ref[0] > 0, 1.0, act)
    s1_ref[...] = s1.reshape(1, 1, _ROWS)
    wout_ref[...] = jnp.clip(w * _DECAY, -2.0, 2.0)


def _sc_body(w_hbm, br_hbm, gpos_hbm, gstart_hbm, gcount_hbm, sperm_hbm,
             sle_hbm, wout_hbm, wrow_v, brrow_v, acca_v, bany_v,
             mpos_v, mstart_v, mcount_v, sperm_v, sle_v):
    def scalar_at(ref, idx):
        return ref[pl.ds(idx, _L)][0]
    lane = jax.lax.iota(jnp.int32, _L)
    wid = jax.lax.axis_index("s") * 2 + jax.lax.axis_index("c")
    base_g = wid * _GPT

    pltpu.sync_copy(gpos_hbm.at[pl.ds(base_g, _GPT)],
                    mpos_v.at[pl.ds(0, _GPT)])
    pltpu.sync_copy(gstart_hbm.at[pl.ds(base_g, _GPT)],
                    mstart_v.at[pl.ds(0, _GPT)])
    pltpu.sync_copy(gcount_hbm.at[pl.ds(base_g, _GPT)],
                    mcount_v.at[pl.ds(0, _GPT)])
    pltpu.sync_copy(sperm_hbm, sperm_v)
    pltpu.sync_copy(sle_hbm, sle_v)

    zeros = jnp.zeros((_L,), jnp.float32)

    def zero_chunk(c, _):
        for u in range(_U):
            sl = pl.ds((c * _U + u) * _L, _L)
            acca_v[sl] = zeros
            bany_v[sl] = zeros
        return 0

    jax.lax.fori_loop(0, _CH // _U, zero_chunk, 0)

    def group_body(k, _):
        pos_k = scalar_at(mpos_v, k)
        start_k = scalar_at(mstart_v, k)
        count_k = scalar_at(mcount_v, k)

        @pl.when(count_k > 0)
        def _():
            pltpu.sync_copy(w_hbm.at[pos_k], wrow_v)

            def am(c, carry):
                bv, bi = carry
                for u in range(_U):
                    cc = c * _U + u
                    ch = wrow_v[pl.ds(cc * _L, _L)]
                    p = jnp.maximum(ch, 0.0) + 1e-6
                    gt = p > bv
                    bv = jnp.where(gt, p, bv)
                    bi = jnp.where(gt, lane + cc * _L, bi)
                return bv, bi

            bv, bi = jax.lax.fori_loop(
                0, _CH // _U, am,
                (jnp.full((_L,), -1.0, jnp.float32),
                 jnp.zeros((_L,), jnp.int32)))
            best_v = bv[0]
            best_i = bi[0]
            for l in range(1, _L):
                v = bv[l]
                i = bi[l]
                better = (v > best_v) | ((v == best_v) & (i < best_i))
                best_v = jnp.where(better, v, best_v)
                best_i = jnp.where(better, i, best_i)
            m = best_i

            def member(j, sumle):
                jj = start_k + j
                pj = scalar_at(sperm_v, jj)
                lej = scalar_at(sle_v, jj)
                pltpu.sync_copy(br_hbm.at[pj], brrow_v)

                def mb(c, _2):
                    for u in range(_U):
                        sl = pl.ds((c * _U + u) * _L, _L)
                        hit = brrow_v[sl] < _BRANCH_PROB
                        acca_v[sl] = acca_v[sl] + jnp.where(hit, lej, 0.0)
                        bany_v[sl] = jnp.maximum(bany_v[sl],
                                                 jnp.where(hit, 1.0, 0.0))
                    return 0

                jax.lax.fori_loop(0, _CH // _U, mb, 0)
                return sumle + lej

            sumle = jax.lax.fori_loop(0, count_k, member, jnp.float32(0.0))

            def fin(c, _2):
                for u in range(_U):
                    cc = c * _U + u
                    sl = pl.ds(cc * _L, _L)
                    w = wrow_v[sl]
                    p = jnp.maximum(w, 0.0) + 1e-6
                    strong = p > _BRANCH_THRESHOLD
                    oh = (lane + cc * _L) == m
                    upd = (jnp.where(strong, acca_v[sl], 0.0)
                           + jnp.where(oh, sumle, 0.0))
                    vis = (strong & (bany_v[sl] > 0.0)) | oh
                    dec = jnp.where(vis, _DECAY_VIS, _DECAY)
                    out = jnp.minimum(jnp.maximum((w + upd) * dec, -2.0),
                                      2.0)
                    wrow_v[sl] = out
                    acca_v[sl] = zeros
                    bany_v[sl] = zeros
                return 0

            jax.lax.fori_loop(0, _CH // _U, fin, 0)
            pltpu.sync_copy(wrow_v, wout_hbm.at[pos_k])

        return 0

    jax.lax.fori_loop(0, _GPT, group_body, 0)


def _sc_update(W, branch_rand, gpos, gstart, gcount, sperm, sle, wref):
    mesh = plsc.VectorSubcoreMesh(core_axis_name="c", subcore_axis_name="s")
    run = pl.kernel(
        _sc_body,
        out_type=(),
        mesh=mesh,
        scratch_types=[
            pltpu.VMEM((_N,), jnp.float32),
            pltpu.VMEM((_N,), jnp.float32),
            pltpu.VMEM((_N,), jnp.float32),
            pltpu.VMEM((_N,), jnp.float32),
            pltpu.VMEM((_GPT + _L,), jnp.int32),
            pltpu.VMEM((_GPT + _L,), jnp.int32),
            pltpu.VMEM((_GPT + _L,), jnp.int32),
            pltpu.VMEM((_S + _L,), jnp.int32),
            pltpu.VMEM((_S + _L,), jnp.float32),
        ],
    )
    run(W, branch_rand, gpos, gstart, gcount, sperm, sle, wref)


def kernel(W, s, noise, spark_pos, spark_energy, spark_age, branch_rand):
    pos32 = spark_pos.astype(jnp.int32)
    order = jnp.argsort(pos32, stable=True).astype(jnp.int32)
    spos = pos32[order]
    change = (spos[1:] != spos[:-1]).astype(jnp.int32)
    one = jnp.ones((1,), jnp.int32)
    first = jnp.concatenate([one, change])
    le = (_LR_PATH * spark_energy)[order]

    gid = jnp.cumsum(first) - 1
    arange_s = jnp.arange(_S, dtype=jnp.int32)
    g_count = jnp.zeros((_S,), jnp.int32).at[gid].add(1)
    g_start = jnp.full((_S,), _S, jnp.int32).at[gid].min(arange_s)
    g_start = jnp.where(g_count > 0, g_start, 0)
    g_pos = jnp.zeros((_S,), jnp.int32).at[gid].max(spos)
    zpad_i = jnp.zeros((_L,), jnp.int32)
    gpos_p = jnp.concatenate([g_pos, zpad_i])
    gstart_p = jnp.concatenate([g_start, zpad_i])
    gcount_p = jnp.concatenate([g_count, zpad_i])
    sperm_p = jnp.concatenate([order, zpad_i])
    sle_p = jnp.concatenate([le, jnp.zeros((_L,), jnp.float32)])

    young = spark_age < _FORCE_STEPS
    force = jnp.zeros((_N,), jnp.float32).at[pos32].set(
        young.astype(jnp.float32))

    s0 = (s * 0.95).reshape(1, _N)
    noise3 = noise.reshape(_NB, 1, _ROWS)
    force3 = force.reshape(_NB, 1, _ROWS)

    wdef, s1 = pl.pallas_call(
        _pass1_body,
        grid=(_NB,),
        in_specs=[
            pl.BlockSpec((1, _N), lambda b: (0, 0)),
            pl.BlockSpec((1, 1, _ROWS), lambda b: (b, 0, 0)),
            pl.BlockSpec((1, 1, _ROWS), lambda b: (b, 0, 0)),
            pl.BlockSpec((_ROWS, _N), lambda b: (b, 0)),
        ],
        out_specs=[
            pl.BlockSpec((_ROWS, _N), lambda b: (b, 0)),
            pl.BlockSpec((1, 1, _ROWS), lambda b: (b, 0, 0)),
        ],
        out_shape=[
            jax.ShapeDtypeStruct((_N, _N), jnp.float32),
            jax.ShapeDtypeStruct((_NB, 1, _ROWS), jnp.float32),
        ],
        compiler_params=pltpu.CompilerParams(
            dimension_semantics=("arbitrary",)),
    )(s0, noise3, force3, W)

    wref = jax.new_ref(wdef)
    _sc_update(W, branch_rand, gpos_p, gstart_p, gcount_p, sperm_p, sle_p,
               wref)
    wout = wref[...]

    return (wout, s1.reshape(_N), spark_energy * _ENERGY_DECAY)
